# baseline (speedup 1.0000x reference)
import math

import jax
import jax.numpy as jnp
from jax import lax
from jax.experimental import pallas as pl
from jax.experimental.pallas import tpu as pltpu
from jax.experimental.pallas import tpu_sc as plsc

VOCAB = 1000000
D = 64
L = 16
NC = 2
NS = 16
NW = NC * NS

CHUNK = 512


def _emb_kernel(n_total: int):
    b_per_w = n_total // NW
    steps = b_per_w // CHUNK
    mesh = plsc.VectorSubcoreMesh(core_axis_name="c", subcore_axis_name="s")

    @pl.kernel(
        out_type=jax.ShapeDtypeStruct((n_total, D), jnp.float32),
        mesh=mesh,
        compiler_params=pltpu.CompilerParams(use_tc_tiling_on_sc=False),
        scratch_types=[
            pltpu.VMEM((b_per_w,), jnp.int32),
            pltpu.VMEM((CHUNK, D), jnp.float32),
            pltpu.SemaphoreType.DMA,
        ],
    )
    def k(idx_hbm, table_hbm, out_hbm, idx_v, rows_v, sem):
        wid = lax.axis_index("s") * NC + lax.axis_index("c")
        base = wid * b_per_w
        pltpu.sync_copy(idx_hbm.at[pl.ds(base, b_per_w)], idx_v)

        @pl.loop(0, steps)
        def _(i):
            pltpu.async_copy(
                table_hbm.at[idx_v.at[pl.ds(i * CHUNK, CHUNK)]], rows_v, sem
            ).wait()

            @pl.loop(0, CHUNK)
            def _(r):
                @pl.loop(0, D, step=L)
                def _(j):
                    rows_v[r, pl.ds(j, L)] = rows_v[r, pl.ds(j, L)] * 8.0

            pltpu.sync_copy(rows_v, out_hbm.at[pl.ds(base + i * CHUNK, CHUNK)])

    return k


def kernel(x, W):
    orig_shape = x.shape
    idx = x.reshape(-1).astype(jnp.int32)
    out = _emb_kernel(idx.shape[0])(idx, W)
    return out.reshape(*orig_shape, D)


# speedup vs baseline: 1.1142x; 1.1142x over previous
import jax
import jax.numpy as jnp
from jax import lax
from jax.experimental import pallas as pl
from jax.experimental.pallas import tpu as pltpu
from jax.experimental.pallas import tpu_sc as plsc

D = 64
L = 16
NC = 2
NS = 16
NW = NC * NS

CHUNK = 256
NBUF = 4


def _emb_kernel(n_total: int):
    b_per_w = n_total // NW
    steps = b_per_w // CHUNK
    assert n_total == NW * CHUNK * steps and steps % NBUF == 0
    mesh = plsc.VectorSubcoreMesh(core_axis_name="c", subcore_axis_name="s")

    @pl.kernel(
        out_type=jax.ShapeDtypeStruct((n_total, D), jnp.float32),
        mesh=mesh,
        compiler_params=pltpu.CompilerParams(use_tc_tiling_on_sc=False),
        scratch_types=[
            pltpu.VMEM((b_per_w,), jnp.int32),
            pltpu.VMEM((NBUF, CHUNK, D), jnp.float32),
        ]
        + [pltpu.SemaphoreType.DMA] * (2 * NBUF),
    )
    def k(idx_hbm, table_hbm, out_hbm, idx_v, rows_v, *sems):
        sg = sems[:NBUF]
        so = sems[NBUF:]
        wid = lax.axis_index("s") * NC + lax.axis_index("c")
        base = wid * b_per_w
        pltpu.sync_copy(idx_hbm.at[pl.ds(base, b_per_w)], idx_v)

        def g_start(j, b):
            pltpu.async_copy(
                table_hbm.at[idx_v.at[pl.ds(j * CHUNK, CHUNK)]], rows_v.at[b], sg[b]
            )

        def g_wait(j, b):
            pltpu.make_async_copy(
                table_hbm.at[idx_v.at[pl.ds(j * CHUNK, CHUNK)]], rows_v.at[b], sg[b]
            ).wait()

        def o_start(j, b):
            pltpu.async_copy(
                rows_v.at[b], out_hbm.at[pl.ds(base + j * CHUNK, CHUNK)], so[b]
            )

        def o_wait(j, b):
            pltpu.make_async_copy(
                rows_v.at[b], out_hbm.at[pl.ds(base + j * CHUNK, CHUNK)], so[b]
            ).wait()

        def scale(b):
            @pl.loop(0, CHUNK, step=4)
            def _(r):
                for rr in range(4):
                    for c0 in range(0, D, L):
                        rows_v[b, r + rr, pl.ds(c0, L)] = (
                            rows_v[b, r + rr, pl.ds(c0, L)] * 8.0
                        )

        g_start(0, 0)
        g_start(1, 1)

        @pl.loop(0, steps, step=NBUF)
        def _(c):
            for u in range(NBUF):
                b = u
                j = c + u
                g_wait(j, b)
                scale(b)
                o_start(j, b)

                @pl.when(j >= 1)
                def _(j=j, u=u):
                    o_wait(j - 1, (u - 1) % NBUF)

                @pl.when(j + 2 < steps)
                def _(j=j, u=u):
                    g_start(j + 2, (u + 2) % NBUF)

        o_wait(steps - 1, (steps - 1) % NBUF)

    return k


def kernel(x, W):
    orig_shape = x.shape
    idx = x.reshape(-1).astype(jnp.int32)
    out = _emb_kernel(idx.shape[0])(idx, W)
    return out.reshape(*orig_shape, D)


# speedup vs baseline: 1.4851x; 1.3329x over previous
import jax
import jax.numpy as jnp
from jax import lax
from jax.experimental import pallas as pl
from jax.experimental.pallas import tpu as pltpu
from jax.experimental.pallas import tpu_sc as plsc

D = 64
L = 16
NC = 2
NS = 16
NW = NC * NS

CHUNK = 256
NBUF = 4


def _emb_kernel(n_total: int):
    b_per_w = n_total // NW
    steps = b_per_w // CHUNK
    assert n_total == NW * CHUNK * steps and steps % NBUF == 0
    mesh = plsc.VectorSubcoreMesh(core_axis_name="c", subcore_axis_name="s")

    @pl.kernel(
        out_type=jax.ShapeDtypeStruct((n_total, 2 * D), jnp.float32),
        mesh=mesh,
        compiler_params=pltpu.CompilerParams(use_tc_tiling_on_sc=False),
        scratch_types=[
            pltpu.VMEM((b_per_w,), jnp.int32),
            pltpu.VMEM((NBUF, CHUNK, D), jnp.float32),
        ]
        + [pltpu.SemaphoreType.DMA] * (2 * NBUF),
    )
    def k(idx_hbm, table_hbm, out_hbm, idx_v, rows_v, *sems):
        sg = sems[:NBUF]
        so = sems[NBUF:]
        wid = lax.axis_index("s") * NC + lax.axis_index("c")
        base = wid * b_per_w
        pltpu.sync_copy(idx_hbm.at[pl.ds(base, b_per_w)], idx_v)

        def g_start(j, b):
            pltpu.async_copy(
                table_hbm.at[idx_v.at[pl.ds(j * CHUNK, CHUNK)]], rows_v.at[b], sg[b]
            )

        def g_wait(j, b):
            pltpu.make_async_copy(
                table_hbm.at[idx_v.at[pl.ds(j * CHUNK, CHUNK)]], rows_v.at[b], sg[b]
            ).wait()

        def o_start(j, b):
            pltpu.async_copy(
                rows_v.at[b],
                out_hbm.at[pl.ds(base + j * CHUNK, CHUNK), pl.ds(0, D)],
                so[b],
            )

        def o_wait(j, b):
            pltpu.make_async_copy(
                rows_v.at[b],
                out_hbm.at[pl.ds(base + j * CHUNK, CHUNK), pl.ds(0, D)],
                so[b],
            ).wait()

        def scale(b):
            @pl.loop(0, CHUNK, step=4)
            def _(r):
                for rr in range(4):
                    for c0 in range(0, D, L):
                        rows_v[b, r + rr, pl.ds(c0, L)] = (
                            rows_v[b, r + rr, pl.ds(c0, L)] * 8.0
                        )

        g_start(0, 0)
        g_start(1, 1)

        @pl.loop(0, steps, step=NBUF)
        def _(c):
            for u in range(NBUF):
                b = u
                j = c + u
                g_wait(j, b)
                scale(b)
                o_start(j, b)

                @pl.when(j >= 1)
                def _(j=j, u=u):
                    o_wait(j - 1, (u - 1) % NBUF)

                @pl.when(j + 2 < steps)
                def _(j=j, u=u):
                    g_start(j + 2, (u + 2) % NBUF)

        o_wait(steps - 1, (steps - 1) % NBUF)

    return k


def kernel(x, W):
    orig_shape = x.shape
    idx = x.reshape(-1).astype(jnp.int32)
    out = _emb_kernel(idx.shape[0])(idx, W)
    return out[:, :D].reshape(*orig_shape, D)
